# baseline (speedup 1.0000x reference)
import functools

import numpy as np
import jax
import jax.numpy as jnp
from jax import lax
from jax.experimental import pallas as pl
from jax.experimental.pallas import tpu as pltpu
from jax.experimental.pallas import tpu_sc as plsc

CUTOFF = 5.0
MOL_NODES = 100
N_REACT = 50
LANES = 16
NCORE = 2
NSUB = 16
NW = NCORE * NSUB
K = 128
KM = 80
TAB = 16384
TABR = 16640


def _ssp(x):
    return jnp.logaddexp(x, 0.0) - jnp.log(2.0)


def _sc_edge_dist2(coord, row, col):
    N = coord.shape[0]
    E = row.shape[0]
    KD = 800
    assert E % KD == 0
    nchunks = E // KD
    per_w = pl.cdiv(nchunks, NW)
    mesh = plsc.VectorSubcoreMesh(core_axis_name="c", subcore_axis_name="s")

    @functools.partial(
        pl.kernel,
        mesh=mesh,
        out_type=jax.ShapeDtypeStruct((E,), jnp.float32),
        scratch_types=[
            pltpu.VMEM((N * 3,), jnp.float32),
            pltpu.VMEM((KD,), jnp.int32),
            pltpu.VMEM((KD,), jnp.int32),
            pltpu.VMEM((KD,), jnp.float32),
        ],
        compiler_params=pltpu.CompilerParams(needs_layout_passes=False),
    )
    def body(coord_hbm, row_hbm, col_hbm, out_hbm, coordv, rowv, colv, d2v):
        c = lax.axis_index("c")
        s = lax.axis_index("s")
        wid = s * NCORE + c
        pltpu.sync_copy(coord_hbm, coordv)

        def chunk_body(j, carry):
            g = j * NW + wid

            @pl.when(g < nchunks)
            def _():
                base = g * KD
                pltpu.sync_copy(row_hbm.at[pl.ds(base, KD)], rowv)
                pltpu.sync_copy(col_hbm.at[pl.ds(base, KD)], colv)
                for jj in range(KD // LANES):
                    ir = rowv[pl.ds(jj * LANES, LANES)] * 3
                    ic = colv[pl.ds(jj * LANES, LANES)] * 3
                    acc = jnp.zeros((LANES,), jnp.float32)
                    for dim in range(3):
                        xr = plsc.load_gather(coordv, [ir + dim])
                        xc = plsc.load_gather(coordv, [ic + dim])
                        d = xr - xc
                        acc = acc + d * d
                    d2v[pl.ds(jj * LANES, LANES)] = acc
                pltpu.sync_copy(d2v, out_hbm.at[pl.ds(base, KD)])

            return carry

        lax.fori_loop(0, per_w, chunk_body, 0)

    return body(coord.reshape(-1), row, col)


def _tc_table(filt_W1, filt_b1, filt_W2, filt_b2):
    NL, NR, F = filt_W1.shape
    BT = 1280
    assert TABR % BT == 0
    step = np.float32(CUTOFF / (NR - 1))
    coeff = np.float32(-0.5 / step**2)
    hk = np.float32(CUTOFF / TAB)

    def body(w1_ref, b1_ref, w2_ref, b2_ref, out_ref):
        t = pl.program_id(0)
        idx = lax.broadcasted_iota(jnp.int32, (BT, 1), 0) + t * BT
        r = idx.astype(jnp.float32) * hk
        offs = lax.broadcasted_iota(jnp.int32, (1, NR), 1).astype(jnp.float32) * step
        rbf = jnp.exp(coeff * (r - offs) ** 2)
        cut = 0.5 * (jnp.cos(r * (np.pi / CUTOFF)) + 1.0)
        cut = cut * (r < CUTOFF).astype(jnp.float32)
        for i in range(NL):
            a = _ssp(jnp.dot(rbf, w1_ref[i], preferred_element_type=jnp.float32)
                     + b1_ref[i][None, :])
            w = jnp.dot(a, w2_ref[i], preferred_element_type=jnp.float32)
            w = w + b2_ref[i][None, :]
            out_ref[i] = w * cut

    return pl.pallas_call(
        body,
        grid=(TABR // BT,),
        in_specs=[
            pl.BlockSpec((NL, NR, F), lambda t: (0, 0, 0)),
            pl.BlockSpec((NL, F), lambda t: (0, 0)),
            pl.BlockSpec((NL, F, F), lambda t: (0, 0, 0)),
            pl.BlockSpec((NL, F), lambda t: (0, 0)),
        ],
        out_specs=pl.BlockSpec((NL, BT, F), lambda t: (0, t, 0)),
        out_shape=jax.ShapeDtypeStruct((NL, TABR, F), jnp.float32),
    )(filt_W1, filt_b1, filt_W2, filt_b2)


def _tc_pack(d2, edge_mask, col, row):
    E = d2.shape[0]
    nchunks = E // KM
    CB = 500
    assert nchunks % CB == 0
    invh = np.float32(TAB / CUTOFF)

    def body(d2_ref, em_ref, col_ref, row_ref, out_ref):
        r = jnp.sqrt(d2_ref[:, 0, :])
        b = jnp.minimum(r * invh + 0.5, np.float32(TABR - 1)).astype(jnp.int32)
        b = jnp.where(em_ref[:, 0, :] == 0.0, TABR - 1, b)
        out_ref[:, 0, :] = col_ref[:, 0, :]
        out_ref[:, 1, :] = b
        out_ref[:, 2, :] = row_ref[:, 0, :]

    return pl.pallas_call(
        body,
        grid=(nchunks // CB,),
        in_specs=[
            pl.BlockSpec((CB, 1, KM), lambda e: (e, 0, 0)),
            pl.BlockSpec((CB, 1, KM), lambda e: (e, 0, 0)),
            pl.BlockSpec((CB, 1, KM), lambda e: (e, 0, 0)),
            pl.BlockSpec((CB, 1, KM), lambda e: (e, 0, 0)),
        ],
        out_specs=pl.BlockSpec((CB, 3, KM), lambda e: (e, 0, 0)),
        out_shape=jax.ShapeDtypeStruct((nchunks, 3, KM), jnp.int32),
    )(d2.reshape(nchunks, 1, KM), edge_mask.reshape(nchunks, 1, KM),
      col.reshape(nchunks, 1, KM), row.reshape(nchunks, 1, KM))


def _tc_embed(z2d, emb, in2f_W0):
    N = z2d.shape[0]
    MAXZ, F = emb.shape
    BN = 2000
    assert N % BN == 0

    def body(z_ref, emb_ref, w_ref, h_ref, xf_ref):
        zb = z_ref[:, 0]
        oh = (zb[:, None] == lax.broadcasted_iota(jnp.int32, (1, MAXZ), 1))
        h = jnp.dot(oh.astype(jnp.float32), emb_ref[...],
                    preferred_element_type=jnp.float32)
        h_ref[...] = h
        xf_ref[...] = jnp.dot(h, w_ref[...], preferred_element_type=jnp.float32)

    return pl.pallas_call(
        body,
        grid=(N // BN,),
        in_specs=[
            pl.BlockSpec((BN, 1), lambda n: (n, 0)),
            pl.BlockSpec((MAXZ, F), lambda n: (0, 0)),
            pl.BlockSpec((F, F), lambda n: (0, 0)),
        ],
        out_specs=[
            pl.BlockSpec((BN, F), lambda n: (n, 0)),
            pl.BlockSpec((BN, F), lambda n: (n, 0)),
        ],
        out_shape=[
            jax.ShapeDtypeStruct((N, F), jnp.float32),
            jax.ShapeDtypeStruct((N, F), jnp.float32),
        ],
    )(z2d, emb, in2f_W0)


def _sc_message(xf, tab, pk, zeros_nf):
    N, F = xf.shape
    E = pk.shape[0] * KM
    assert E % KM == 0 and N % 8 == 0
    nchunks = E // KM
    per_w = pl.cdiv(nchunks, NW)
    bases = [-(-(s * N // NSUB) // 8) * 8 for s in range(NSUB)] + [N]
    sizes = [bases[s + 1] - bases[s] for s in range(NSUB)]
    mesh = plsc.VectorSubcoreMesh(core_axis_name="c", subcore_axis_name="s")

    @functools.partial(
        pl.kernel,
        mesh=mesh,
        out_type=jax.ShapeDtypeStruct((NCORE, N, F), jnp.float32),
        scratch_types=[
            pltpu.VMEM((2, 3, KM), jnp.int32),
            pltpu.VMEM((2, KM, F), jnp.float32),
            pltpu.VMEM((2, KM, F), jnp.float32),
            pltpu.VMEM_SHARED((N, F), jnp.float32),
            pltpu.SemaphoreType.DMA,
            pltpu.SemaphoreType.DMA,
            pltpu.SemaphoreType.DMA,
            pltpu.SemaphoreType.DMA,
            pltpu.SemaphoreType.DMA,
            pltpu.SemaphoreType.DMA,
        ],
    )
    def body(xf_hbm, tab_hbm, pk_hbm, zeros_hbm, out_hbm,
             pkv, xjv, wijv, acc, sw0, sw1, sg0, sg1, ss0, ss1):
        sem_w = (sw0, sw1)
        sem_g = (sg0, sg1)
        sem_s = (ss0, ss1)
        c = lax.axis_index("c")
        s = lax.axis_index("s")
        wid = s * NCORE + c
        for ss in range(NSUB):
            @pl.when(s == ss)
            def _():
                pltpu.sync_copy(zeros_hbm.at[pl.ds(bases[ss], sizes[ss])],
                                acc.at[pl.ds(bases[ss], sizes[ss])])
        plsc.subcore_barrier()

        def issue(g, b):
            @pl.when(g >= wid + 2 * NW)
            def _():
                pltpu.make_async_copy(xjv.at[b], acc.at[pkv.at[b, 2]],
                                      sem_s[b]).wait()

            pltpu.sync_copy(pk_hbm.at[g], pkv.at[b])
            pltpu.async_copy(tab_hbm.at[pkv.at[b, 1]], wijv.at[b], sem_w[b])
            pltpu.async_copy(xf_hbm.at[pkv.at[b, 0]], xjv.at[b], sem_g[b])

        issue(wid, 0)

        def pair_body(j2, carry):
            for b in range(2):
                j = j2 * 2 + b
                g = j * NW + wid

                @pl.when(g < nchunks)
                def _():
                    pltpu.make_async_copy(
                        tab_hbm.at[pkv.at[b, 1]], wijv.at[b],
                        sem_w[b]).wait()
                    pltpu.make_async_copy(
                        xf_hbm.at[pkv.at[b, 0]], xjv.at[b], sem_g[b]).wait()
                    gn = g + NW

                    @pl.when(gn < nchunks)
                    def _():
                        issue(gn, 1 - b)

                    def mul_body(k, mc):
                        for jj in range(F // LANES):
                            sl = pl.ds(jj * LANES, LANES)
                            xjv[b, k, sl] = xjv[b, k, sl] * wijv[b, k, sl]
                        return mc

                    lax.fori_loop(0, KM, mul_body, 0)
                    pltpu.async_copy(xjv.at[b], acc.at[pkv.at[b, 2]],
                                     sem_s[b], add=True)

            return carry

        lax.fori_loop(0, pl.cdiv(per_w, 2), pair_body, 0)
        for b in range(2):
            pltpu.make_async_copy(xjv.at[b], acc.at[pkv.at[b, 2]],
                                  sem_s[b]).wait()
        plsc.subcore_barrier()
        for ss in range(NSUB):
            @pl.when(s == ss)
            def _():
                pltpu.sync_copy(acc.at[pl.ds(bases[ss], sizes[ss])],
                                out_hbm.at[c, pl.ds(bases[ss], sizes[ss])])

    return body(xf, tab, pk, zeros_nf)


def _tc_update(h, mp, W1, b1, W2, b2, in2f_next):
    N, F = h.shape
    BN = 2000
    assert N % BN == 0

    def body(h_ref, mp_ref, w1_ref, b1_ref, w2_ref, b2_ref, wn_ref,
             hn_ref, xf_ref):
        m = mp_ref[0] + mp_ref[1]
        t = _ssp(jnp.dot(m, w1_ref[...], preferred_element_type=jnp.float32)
                 + b1_ref[0][None, :])
        t = jnp.dot(t, w2_ref[...], preferred_element_type=jnp.float32)
        t = t + b2_ref[0][None, :]
        hn = h_ref[...] + t
        hn_ref[...] = hn
        xf_ref[...] = jnp.dot(hn, wn_ref[...], preferred_element_type=jnp.float32)

    return pl.pallas_call(
        body,
        grid=(N // BN,),
        in_specs=[
            pl.BlockSpec((BN, F), lambda n: (n, 0)),
            pl.BlockSpec((NCORE, BN, F), lambda n: (0, n, 0)),
            pl.BlockSpec((F, F), lambda n: (0, 0)),
            pl.BlockSpec((1, F), lambda n: (0, 0)),
            pl.BlockSpec((F, F), lambda n: (0, 0)),
            pl.BlockSpec((1, F), lambda n: (0, 0)),
            pl.BlockSpec((F, F), lambda n: (0, 0)),
        ],
        out_specs=[
            pl.BlockSpec((BN, F), lambda n: (n, 0)),
            pl.BlockSpec((BN, F), lambda n: (n, 0)),
        ],
        out_shape=[
            jax.ShapeDtypeStruct((N, F), jnp.float32),
            jax.ShapeDtypeStruct((N, F), jnp.float32),
        ],
    )(h, mp, W1, b1.reshape(1, -1), W2, b2.reshape(1, -1), in2f_next)


def _tc_update_last(h, mp, W1, b1, W2, b2, atom_mask, node_delta):
    N, F = h.shape
    BN = 2000
    assert N % BN == 0 and BN % MOL_NODES == 0
    mols_per_blk = BN // MOL_NODES

    def body(h_ref, mp_ref, w1_ref, b1_ref, w2_ref, b2_ref, am_ref, nd_ref,
             hm_ref):
        m = mp_ref[0] + mp_ref[1]
        t = _ssp(jnp.dot(m, w1_ref[...], preferred_element_type=jnp.float32)
                 + b1_ref[0][None, :])
        t = jnp.dot(t, w2_ref[...], preferred_element_type=jnp.float32)
        t = t + b2_ref[0][None, :]
        hn = (h_ref[...] + t) * am_ref[...] + nd_ref[0, 0]
        hm_ref[...] = jnp.sum(hn.reshape(mols_per_blk, MOL_NODES, F),
                              axis=1)[None]

    return pl.pallas_call(
        body,
        grid=(N // BN,),
        in_specs=[
            pl.BlockSpec((BN, F), lambda n: (n, 0)),
            pl.BlockSpec((NCORE, BN, F), lambda n: (0, n, 0)),
            pl.BlockSpec((F, F), lambda n: (0, 0)),
            pl.BlockSpec((1, F), lambda n: (0, 0)),
            pl.BlockSpec((F, F), lambda n: (0, 0)),
            pl.BlockSpec((1, F), lambda n: (0, 0)),
            pl.BlockSpec((BN, 1), lambda n: (n, 0)),
            pl.BlockSpec((1, 1), lambda n: (0, 0)),
        ],
        out_specs=pl.BlockSpec((1, mols_per_blk, F), lambda n: (n, 0, 0)),
        out_shape=jax.ShapeDtypeStruct((N // BN, mols_per_blk, F),
                                       jnp.float32),
    )(h, mp, W1, b1.reshape(1, -1), W2, b2.reshape(1, -1), atom_mask,
      node_delta).reshape(N // MOL_NODES, F)


def _tc_readout(hmol, dec_W1, dec_b1, dec_W2, dec_b2, rid2d, sgn2d,
                react_delta):
    B, F = hmol.shape
    H = dec_W1.shape[1]

    def body(hm_ref, w1_ref, b1_ref, w2_ref, b2_ref, rid_ref, sgn_ref,
             rd_ref, out_ref):
        t = _ssp(jnp.dot(hm_ref[...], w1_ref[...],
                         preferred_element_type=jnp.float32)
                 + b1_ref[0][None, :])
        t = jnp.dot(t, w2_ref[...], preferred_element_type=jnp.float32)
        t = t + b2_ref[0][None, :]
        t = t * sgn_ref[...]
        sel = (rid_ref[...] == lax.broadcasted_iota(jnp.int32, (1, N_REACT), 1))
        contrib = sel.astype(jnp.float32) * t
        out_ref[...] = jnp.sum(contrib, axis=0, keepdims=True) + rd_ref[0, 0]

    return pl.pallas_call(
        body,
        grid=(1,),
        in_specs=[
            pl.BlockSpec((B, F), lambda i: (0, 0)),
            pl.BlockSpec((F, H), lambda i: (0, 0)),
            pl.BlockSpec((1, H), lambda i: (0, 0)),
            pl.BlockSpec((H, 1), lambda i: (0, 0)),
            pl.BlockSpec((1, 1), lambda i: (0, 0)),
            pl.BlockSpec((B, 1), lambda i: (0, 0)),
            pl.BlockSpec((B, 1), lambda i: (0, 0)),
            pl.BlockSpec((1, 1), lambda i: (0, 0)),
        ],
        out_specs=pl.BlockSpec((1, N_REACT), lambda i: (0, 0)),
        out_shape=jax.ShapeDtypeStruct((1, N_REACT), jnp.float32),
    )(hmol, dec_W1, dec_b1.reshape(1, -1), dec_W2, dec_b2.reshape(1, -1),
      rid2d, sgn2d, react_delta)


def kernel(z, coord, edge_index, n_nodes, atom_mask, edge_mask, n_reactions,
           reaction_indexes, reaction_indexes_signs, emb, in2f_W, filt_W1,
           filt_b1, filt_W2, filt_b2, f2out_W1, f2out_b1, f2out_W2, f2out_b2,
           dec_W1, dec_b1, dec_W2, dec_b2):
    row = edge_index[0].astype(jnp.int32)
    col = edge_index[1].astype(jnp.int32)
    N, F = coord.shape[0], emb.shape[1]
    NL = in2f_W.shape[0]

    d2 = _sc_edge_dist2(coord, row, col)
    tab = _tc_table(filt_W1, filt_b1, filt_W2, filt_b2)
    pk = _tc_pack(d2, edge_mask, col, row)
    h, xf = _tc_embed(z.astype(jnp.int32).reshape(-1, 1), emb, in2f_W[0])
    zeros_nf = jnp.zeros((N, F), jnp.float32)

    node_delta = (jnp.asarray(n_nodes, jnp.float32)
                  - np.float32(MOL_NODES)).reshape(1, 1)
    react_delta = (jnp.asarray(n_reactions, jnp.float32)
                   - np.float32(N_REACT)).reshape(1, 1)

    hmol = None
    for i in range(NL):
        mp = _sc_message(xf, tab[i], pk, zeros_nf)
        if i + 1 < NL:
            h, xf = _tc_update(h, mp, f2out_W1[i], f2out_b1[i], f2out_W2[i],
                               f2out_b2[i], in2f_W[i + 1])
        else:
            hmol = _tc_update_last(h, mp, f2out_W1[i], f2out_b1[i],
                                   f2out_W2[i], f2out_b2[i], atom_mask,
                                   node_delta)

    rid2d = reaction_indexes.astype(jnp.int32).reshape(-1, 1)
    sgn2d = reaction_indexes_signs.reshape(-1, 1)
    pred = _tc_readout(hmol, dec_W1, dec_b1, dec_W2, dec_b2, rid2d, sgn2d,
                       react_delta)
    return pred[0]


# speedup vs baseline: 1.0947x; 1.0947x over previous
import functools

import numpy as np
import jax
import jax.numpy as jnp
from jax import lax
from jax.experimental import pallas as pl
from jax.experimental.pallas import tpu as pltpu
from jax.experimental.pallas import tpu_sc as plsc

CUTOFF = 5.0
MOL_NODES = 100
N_REACT = 50
LANES = 16
NCORE = 2
NSUB = 16
NW = NCORE * NSUB
K = 128
KM = 64
TAB = 16384
TABR = 16640


def _ssp(x):
    return jnp.logaddexp(x, 0.0) - jnp.log(2.0)


def _sc_edge_dist2(coord, row, col):
    N = coord.shape[0]
    E = row.shape[0]
    KD = 800
    assert E % KD == 0
    nchunks = E // KD
    per_w = pl.cdiv(nchunks, NW)
    mesh = plsc.VectorSubcoreMesh(core_axis_name="c", subcore_axis_name="s")

    @functools.partial(
        pl.kernel,
        mesh=mesh,
        out_type=jax.ShapeDtypeStruct((E,), jnp.float32),
        scratch_types=[
            pltpu.VMEM((N * 3,), jnp.float32),
            pltpu.VMEM((KD,), jnp.int32),
            pltpu.VMEM((KD,), jnp.int32),
            pltpu.VMEM((KD,), jnp.float32),
        ],
        compiler_params=pltpu.CompilerParams(needs_layout_passes=False),
    )
    def body(coord_hbm, row_hbm, col_hbm, out_hbm, coordv, rowv, colv, d2v):
        c = lax.axis_index("c")
        s = lax.axis_index("s")
        wid = s * NCORE + c
        pltpu.sync_copy(coord_hbm, coordv)

        def chunk_body(j, carry):
            g = j * NW + wid

            @pl.when(g < nchunks)
            def _():
                base = g * KD
                pltpu.sync_copy(row_hbm.at[pl.ds(base, KD)], rowv)
                pltpu.sync_copy(col_hbm.at[pl.ds(base, KD)], colv)
                for jj in range(KD // LANES):
                    ir = rowv[pl.ds(jj * LANES, LANES)] * 3
                    ic = colv[pl.ds(jj * LANES, LANES)] * 3
                    acc = jnp.zeros((LANES,), jnp.float32)
                    for dim in range(3):
                        xr = plsc.load_gather(coordv, [ir + dim])
                        xc = plsc.load_gather(coordv, [ic + dim])
                        d = xr - xc
                        acc = acc + d * d
                    d2v[pl.ds(jj * LANES, LANES)] = acc
                pltpu.sync_copy(d2v, out_hbm.at[pl.ds(base, KD)])

            return carry

        lax.fori_loop(0, per_w, chunk_body, 0)

    return body(coord.reshape(-1), row, col)


def _tc_table(filt_W1, filt_b1, filt_W2, filt_b2):
    NL, NR, F = filt_W1.shape
    BT = 1280
    assert TABR % BT == 0
    step = np.float32(CUTOFF / (NR - 1))
    coeff = np.float32(-0.5 / step**2)
    hk = np.float32(CUTOFF / TAB)

    def body(w1_ref, b1_ref, w2_ref, b2_ref, out_ref):
        t = pl.program_id(0)
        idx = lax.broadcasted_iota(jnp.int32, (BT, 1), 0) + t * BT
        r = idx.astype(jnp.float32) * hk
        offs = lax.broadcasted_iota(jnp.int32, (1, NR), 1).astype(jnp.float32) * step
        rbf = jnp.exp(coeff * (r - offs) ** 2)
        cut = 0.5 * (jnp.cos(r * (np.pi / CUTOFF)) + 1.0)
        cut = cut * (r < CUTOFF).astype(jnp.float32)
        for i in range(NL):
            a = _ssp(jnp.dot(rbf, w1_ref[i], preferred_element_type=jnp.float32)
                     + b1_ref[i][None, :])
            w = jnp.dot(a, w2_ref[i], preferred_element_type=jnp.float32)
            w = w + b2_ref[i][None, :]
            out_ref[i] = w * cut

    return pl.pallas_call(
        body,
        grid=(TABR // BT,),
        in_specs=[
            pl.BlockSpec((NL, NR, F), lambda t: (0, 0, 0)),
            pl.BlockSpec((NL, F), lambda t: (0, 0)),
            pl.BlockSpec((NL, F, F), lambda t: (0, 0, 0)),
            pl.BlockSpec((NL, F), lambda t: (0, 0)),
        ],
        out_specs=pl.BlockSpec((NL, BT, F), lambda t: (0, t, 0)),
        out_shape=jax.ShapeDtypeStruct((NL, TABR, F), jnp.float32),
    )(filt_W1, filt_b1, filt_W2, filt_b2)


def _tc_pack(d2, edge_mask, col, row):
    E = d2.shape[0]
    nchunks = E // KM
    CB = 500
    assert nchunks % CB == 0
    invh = np.float32(TAB / CUTOFF)

    def body(d2_ref, em_ref, col_ref, row_ref, out_ref):
        r = jnp.sqrt(d2_ref[:, 0, :])
        b = jnp.minimum(r * invh + 0.5, np.float32(TABR - 1)).astype(jnp.int32)
        b = jnp.where(em_ref[:, 0, :] == 0.0, TABR - 1, b)
        out_ref[:, 0, :] = col_ref[:, 0, :]
        out_ref[:, 1, :] = b
        out_ref[:, 2, :] = row_ref[:, 0, :]

    return pl.pallas_call(
        body,
        grid=(nchunks // CB,),
        in_specs=[
            pl.BlockSpec((CB, 1, KM), lambda e: (e, 0, 0)),
            pl.BlockSpec((CB, 1, KM), lambda e: (e, 0, 0)),
            pl.BlockSpec((CB, 1, KM), lambda e: (e, 0, 0)),
            pl.BlockSpec((CB, 1, KM), lambda e: (e, 0, 0)),
        ],
        out_specs=pl.BlockSpec((CB, 3, KM), lambda e: (e, 0, 0)),
        out_shape=jax.ShapeDtypeStruct((nchunks, 3, KM), jnp.int32),
    )(d2.reshape(nchunks, 1, KM), edge_mask.reshape(nchunks, 1, KM),
      col.reshape(nchunks, 1, KM), row.reshape(nchunks, 1, KM))


def _tc_embed(z2d, emb, in2f_W0):
    N = z2d.shape[0]
    MAXZ, F = emb.shape
    BN = 2000
    assert N % BN == 0

    def body(z_ref, emb_ref, w_ref, h_ref, xf_ref):
        zb = z_ref[:, 0]
        oh = (zb[:, None] == lax.broadcasted_iota(jnp.int32, (1, MAXZ), 1))
        h = jnp.dot(oh.astype(jnp.float32), emb_ref[...],
                    preferred_element_type=jnp.float32)
        h_ref[...] = h
        xf_ref[...] = jnp.dot(h, w_ref[...], preferred_element_type=jnp.float32)

    return pl.pallas_call(
        body,
        grid=(N // BN,),
        in_specs=[
            pl.BlockSpec((BN, 1), lambda n: (n, 0)),
            pl.BlockSpec((MAXZ, F), lambda n: (0, 0)),
            pl.BlockSpec((F, F), lambda n: (0, 0)),
        ],
        out_specs=[
            pl.BlockSpec((BN, F), lambda n: (n, 0)),
            pl.BlockSpec((BN, F), lambda n: (n, 0)),
        ],
        out_shape=[
            jax.ShapeDtypeStruct((N, F), jnp.float32),
            jax.ShapeDtypeStruct((N, F), jnp.float32),
        ],
    )(z2d, emb, in2f_W0)


def _sc_message(xf, tab, pk, zeros_nf):
    N, F = xf.shape
    E = pk.shape[0] * KM
    assert E % KM == 0 and N % 8 == 0
    nchunks = E // KM
    per_w = pl.cdiv(nchunks, NW)
    bases = [-(-(s * N // NSUB) // 8) * 8 for s in range(NSUB)] + [N]
    sizes = [bases[s + 1] - bases[s] for s in range(NSUB)]
    mesh = plsc.VectorSubcoreMesh(core_axis_name="c", subcore_axis_name="s")

    @functools.partial(
        pl.kernel,
        mesh=mesh,
        out_type=jax.ShapeDtypeStruct((NCORE, N, F), jnp.float32),
        scratch_types=[
            pltpu.VMEM((3, 3, KM), jnp.int32),
            pltpu.VMEM((3, KM, F), jnp.float32),
            pltpu.VMEM((3, KM, F), jnp.float32),
            pltpu.VMEM_SHARED((N, F), jnp.float32),
            [pltpu.SemaphoreType.DMA] * 3,
            [pltpu.SemaphoreType.DMA] * 3,
            [pltpu.SemaphoreType.DMA] * 3,
        ],
    )
    def body(xf_hbm, tab_hbm, pk_hbm, zeros_hbm, out_hbm,
             pkv, xjv, wijv, acc, sem_w, sem_g, sem_s):
        c = lax.axis_index("c")
        s = lax.axis_index("s")
        wid = s * NCORE + c
        for ss in range(NSUB):
            @pl.when(s == ss)
            def _():
                pltpu.sync_copy(zeros_hbm.at[pl.ds(bases[ss], sizes[ss])],
                                acc.at[pl.ds(bases[ss], sizes[ss])])
        plsc.subcore_barrier()

        def issue(g, b):
            @pl.when(g >= wid + 3 * NW)
            def _():
                pltpu.make_async_copy(xjv.at[b], acc.at[pkv.at[b, 2]],
                                      sem_s[b]).wait()

            pltpu.sync_copy(pk_hbm.at[g], pkv.at[b])
            pltpu.async_copy(tab_hbm.at[pkv.at[b, 1]], wijv.at[b], sem_w[b])
            pltpu.async_copy(xf_hbm.at[pkv.at[b, 0]], xjv.at[b], sem_g[b])

        issue(wid, 0)
        issue(wid + NW, 1)

        def triple_body(j3, carry):
            for b3 in range(3):
                j = j3 * 3 + b3
                b = b3
                g = j * NW + wid

                @pl.when(g < nchunks)
                def _():
                    pltpu.make_async_copy(
                        tab_hbm.at[pkv.at[b, 1]], wijv.at[b],
                        sem_w[b]).wait()
                    pltpu.make_async_copy(
                        xf_hbm.at[pkv.at[b, 0]], xjv.at[b], sem_g[b]).wait()
                    gn = g + 2 * NW

                    @pl.when(gn < nchunks)
                    def _():
                        issue(gn, (b + 2) % 3)

                    def mul_body(k, mc):
                        for jj in range(F // LANES):
                            sl = pl.ds(jj * LANES, LANES)
                            xjv[b, k, sl] = xjv[b, k, sl] * wijv[b, k, sl]
                        return mc

                    lax.fori_loop(0, KM, mul_body, 0)
                    pltpu.async_copy(xjv.at[b], acc.at[pkv.at[b, 2]],
                                     sem_s[b], add=True)

            return carry

        lax.fori_loop(0, pl.cdiv(per_w, 3), triple_body, 0)
        for b in range(3):
            pltpu.make_async_copy(xjv.at[b], acc.at[pkv.at[b, 2]],
                                  sem_s[b]).wait()
        plsc.subcore_barrier()
        for ss in range(NSUB):
            @pl.when(s == ss)
            def _():
                pltpu.sync_copy(acc.at[pl.ds(bases[ss], sizes[ss])],
                                out_hbm.at[c, pl.ds(bases[ss], sizes[ss])])

    return body(xf, tab, pk, zeros_nf)


def _tc_update(h, mp, W1, b1, W2, b2, in2f_next):
    N, F = h.shape
    BN = 2000
    assert N % BN == 0

    def body(h_ref, mp_ref, w1_ref, b1_ref, w2_ref, b2_ref, wn_ref,
             hn_ref, xf_ref):
        m = mp_ref[0] + mp_ref[1]
        t = _ssp(jnp.dot(m, w1_ref[...], preferred_element_type=jnp.float32)
                 + b1_ref[0][None, :])
        t = jnp.dot(t, w2_ref[...], preferred_element_type=jnp.float32)
        t = t + b2_ref[0][None, :]
        hn = h_ref[...] + t
        hn_ref[...] = hn
        xf_ref[...] = jnp.dot(hn, wn_ref[...], preferred_element_type=jnp.float32)

    return pl.pallas_call(
        body,
        grid=(N // BN,),
        in_specs=[
            pl.BlockSpec((BN, F), lambda n: (n, 0)),
            pl.BlockSpec((NCORE, BN, F), lambda n: (0, n, 0)),
            pl.BlockSpec((F, F), lambda n: (0, 0)),
            pl.BlockSpec((1, F), lambda n: (0, 0)),
            pl.BlockSpec((F, F), lambda n: (0, 0)),
            pl.BlockSpec((1, F), lambda n: (0, 0)),
            pl.BlockSpec((F, F), lambda n: (0, 0)),
        ],
        out_specs=[
            pl.BlockSpec((BN, F), lambda n: (n, 0)),
            pl.BlockSpec((BN, F), lambda n: (n, 0)),
        ],
        out_shape=[
            jax.ShapeDtypeStruct((N, F), jnp.float32),
            jax.ShapeDtypeStruct((N, F), jnp.float32),
        ],
    )(h, mp, W1, b1.reshape(1, -1), W2, b2.reshape(1, -1), in2f_next)


def _tc_update_last(h, mp, W1, b1, W2, b2, atom_mask, node_delta):
    N, F = h.shape
    BN = 2000
    assert N % BN == 0 and BN % MOL_NODES == 0
    mols_per_blk = BN // MOL_NODES

    def body(h_ref, mp_ref, w1_ref, b1_ref, w2_ref, b2_ref, am_ref, nd_ref,
             hm_ref):
        m = mp_ref[0] + mp_ref[1]
        t = _ssp(jnp.dot(m, w1_ref[...], preferred_element_type=jnp.float32)
                 + b1_ref[0][None, :])
        t = jnp.dot(t, w2_ref[...], preferred_element_type=jnp.float32)
        t = t + b2_ref[0][None, :]
        hn = (h_ref[...] + t) * am_ref[...] + nd_ref[0, 0]
        hm_ref[...] = jnp.sum(hn.reshape(mols_per_blk, MOL_NODES, F),
                              axis=1)[None]

    return pl.pallas_call(
        body,
        grid=(N // BN,),
        in_specs=[
            pl.BlockSpec((BN, F), lambda n: (n, 0)),
            pl.BlockSpec((NCORE, BN, F), lambda n: (0, n, 0)),
            pl.BlockSpec((F, F), lambda n: (0, 0)),
            pl.BlockSpec((1, F), lambda n: (0, 0)),
            pl.BlockSpec((F, F), lambda n: (0, 0)),
            pl.BlockSpec((1, F), lambda n: (0, 0)),
            pl.BlockSpec((BN, 1), lambda n: (n, 0)),
            pl.BlockSpec((1, 1), lambda n: (0, 0)),
        ],
        out_specs=pl.BlockSpec((1, mols_per_blk, F), lambda n: (n, 0, 0)),
        out_shape=jax.ShapeDtypeStruct((N // BN, mols_per_blk, F),
                                       jnp.float32),
    )(h, mp, W1, b1.reshape(1, -1), W2, b2.reshape(1, -1), atom_mask,
      node_delta).reshape(N // MOL_NODES, F)


def _tc_readout(hmol, dec_W1, dec_b1, dec_W2, dec_b2, rid2d, sgn2d,
                react_delta):
    B, F = hmol.shape
    H = dec_W1.shape[1]

    def body(hm_ref, w1_ref, b1_ref, w2_ref, b2_ref, rid_ref, sgn_ref,
             rd_ref, out_ref):
        t = _ssp(jnp.dot(hm_ref[...], w1_ref[...],
                         preferred_element_type=jnp.float32)
                 + b1_ref[0][None, :])
        t = jnp.dot(t, w2_ref[...], preferred_element_type=jnp.float32)
        t = t + b2_ref[0][None, :]
        t = t * sgn_ref[...]
        sel = (rid_ref[...] == lax.broadcasted_iota(jnp.int32, (1, N_REACT), 1))
        contrib = sel.astype(jnp.float32) * t
        out_ref[...] = jnp.sum(contrib, axis=0, keepdims=True) + rd_ref[0, 0]

    return pl.pallas_call(
        body,
        grid=(1,),
        in_specs=[
            pl.BlockSpec((B, F), lambda i: (0, 0)),
            pl.BlockSpec((F, H), lambda i: (0, 0)),
            pl.BlockSpec((1, H), lambda i: (0, 0)),
            pl.BlockSpec((H, 1), lambda i: (0, 0)),
            pl.BlockSpec((1, 1), lambda i: (0, 0)),
            pl.BlockSpec((B, 1), lambda i: (0, 0)),
            pl.BlockSpec((B, 1), lambda i: (0, 0)),
            pl.BlockSpec((1, 1), lambda i: (0, 0)),
        ],
        out_specs=pl.BlockSpec((1, N_REACT), lambda i: (0, 0)),
        out_shape=jax.ShapeDtypeStruct((1, N_REACT), jnp.float32),
    )(hmol, dec_W1, dec_b1.reshape(1, -1), dec_W2, dec_b2.reshape(1, -1),
      rid2d, sgn2d, react_delta)


def kernel(z, coord, edge_index, n_nodes, atom_mask, edge_mask, n_reactions,
           reaction_indexes, reaction_indexes_signs, emb, in2f_W, filt_W1,
           filt_b1, filt_W2, filt_b2, f2out_W1, f2out_b1, f2out_W2, f2out_b2,
           dec_W1, dec_b1, dec_W2, dec_b2):
    row = edge_index[0].astype(jnp.int32)
    col = edge_index[1].astype(jnp.int32)
    N, F = coord.shape[0], emb.shape[1]
    NL = in2f_W.shape[0]

    d2 = _sc_edge_dist2(coord, row, col)
    tab = _tc_table(filt_W1, filt_b1, filt_W2, filt_b2)
    pk = _tc_pack(d2, edge_mask, col, row)
    h, xf = _tc_embed(z.astype(jnp.int32).reshape(-1, 1), emb, in2f_W[0])
    zeros_nf = jnp.zeros((N, F), jnp.float32)

    node_delta = (jnp.asarray(n_nodes, jnp.float32)
                  - np.float32(MOL_NODES)).reshape(1, 1)
    react_delta = (jnp.asarray(n_reactions, jnp.float32)
                   - np.float32(N_REACT)).reshape(1, 1)

    hmol = None
    for i in range(NL):
        mp = _sc_message(xf, tab[i], pk, zeros_nf)
        if i + 1 < NL:
            h, xf = _tc_update(h, mp, f2out_W1[i], f2out_b1[i], f2out_W2[i],
                               f2out_b2[i], in2f_W[i + 1])
        else:
            hmol = _tc_update_last(h, mp, f2out_W1[i], f2out_b1[i],
                                   f2out_W2[i], f2out_b2[i], atom_mask,
                                   node_delta)

    rid2d = reaction_indexes.astype(jnp.int32).reshape(-1, 1)
    sgn2d = reaction_indexes_signs.reshape(-1, 1)
    pred = _tc_readout(hmol, dec_W1, dec_b1, dec_W2, dec_b2, rid2d, sgn2d,
                       react_delta)
    return pred[0]


# speedup vs baseline: 1.0956x; 1.0009x over previous
import functools

import numpy as np
import jax
import jax.numpy as jnp
from jax import lax
from jax.experimental import pallas as pl
from jax.experimental.pallas import tpu as pltpu
from jax.experimental.pallas import tpu_sc as plsc

CUTOFF = 5.0
MOL_NODES = 100
N_REACT = 50
LANES = 16
NCORE = 2
NSUB = 16
NW = NCORE * NSUB
KM = 64
TAB = 16384
TABR = 16640


def _ssp(x):
    return jnp.logaddexp(x, 0.0) - jnp.log(2.0)


def _sc_edge_dist2(coord, row, col):
    N = coord.shape[0]
    E = row.shape[0]
    KD = 800
    assert E % KD == 0
    nchunks = E // KD
    per_w = pl.cdiv(nchunks, NW)
    mesh = plsc.VectorSubcoreMesh(core_axis_name="c", subcore_axis_name="s")

    @functools.partial(
        pl.kernel,
        mesh=mesh,
        out_type=jax.ShapeDtypeStruct((E,), jnp.float32),
        scratch_types=[
            pltpu.VMEM((N * 3,), jnp.float32),
            pltpu.VMEM((KD,), jnp.int32),
            pltpu.VMEM((KD,), jnp.int32),
            pltpu.VMEM((KD,), jnp.float32),
        ],
        compiler_params=pltpu.CompilerParams(needs_layout_passes=False),
    )
    def body(coord_hbm, row_hbm, col_hbm, out_hbm, coordv, rowv, colv, d2v):
        c = lax.axis_index("c")
        s = lax.axis_index("s")
        wid = s * NCORE + c
        pltpu.sync_copy(coord_hbm, coordv)

        def chunk_body(j, carry):
            g = j * NW + wid

            @pl.when(g < nchunks)
            def _():
                base = g * KD
                pltpu.sync_copy(row_hbm.at[pl.ds(base, KD)], rowv)
                pltpu.sync_copy(col_hbm.at[pl.ds(base, KD)], colv)
                for jj in range(KD // LANES):
                    ir = rowv[pl.ds(jj * LANES, LANES)] * 3
                    ic = colv[pl.ds(jj * LANES, LANES)] * 3
                    acc = jnp.zeros((LANES,), jnp.float32)
                    for dim in range(3):
                        xr = plsc.load_gather(coordv, [ir + dim])
                        xc = plsc.load_gather(coordv, [ic + dim])
                        d = xr - xc
                        acc = acc + d * d
                    d2v[pl.ds(jj * LANES, LANES)] = acc
                pltpu.sync_copy(d2v, out_hbm.at[pl.ds(base, KD)])

            return carry

        lax.fori_loop(0, per_w, chunk_body, 0)

    return body(coord.reshape(-1), row, col)


def _tc_table(filt_W1, filt_b1, filt_W2, filt_b2):
    NL, NR, F = filt_W1.shape
    BT = 1280
    assert TABR % BT == 0
    step = np.float32(CUTOFF / (NR - 1))
    coeff = np.float32(-0.5 / step**2)
    hk = np.float32(CUTOFF / TAB)

    def body(w1_ref, b1_ref, w2_ref, b2_ref, out_ref):
        t = pl.program_id(0)
        idx = lax.broadcasted_iota(jnp.int32, (BT, 1), 0) + t * BT
        r = idx.astype(jnp.float32) * hk
        offs = lax.broadcasted_iota(jnp.int32, (1, NR), 1).astype(jnp.float32) * step
        rbf = jnp.exp(coeff * (r - offs) ** 2)
        cut = 0.5 * (jnp.cos(r * (np.pi / CUTOFF)) + 1.0)
        cut = cut * (r < CUTOFF).astype(jnp.float32)
        for i in range(NL):
            a = _ssp(jnp.dot(rbf, w1_ref[i], preferred_element_type=jnp.float32)
                     + b1_ref[i][None, :])
            w = jnp.dot(a, w2_ref[i], preferred_element_type=jnp.float32)
            w = w + b2_ref[i][None, :]
            out_ref[i] = w * cut

    return pl.pallas_call(
        body,
        grid=(TABR // BT,),
        in_specs=[
            pl.BlockSpec((NL, NR, F), lambda t: (0, 0, 0)),
            pl.BlockSpec((NL, F), lambda t: (0, 0)),
            pl.BlockSpec((NL, F, F), lambda t: (0, 0, 0)),
            pl.BlockSpec((NL, F), lambda t: (0, 0)),
        ],
        out_specs=pl.BlockSpec((NL, BT, F), lambda t: (0, t, 0)),
        out_shape=jax.ShapeDtypeStruct((NL, TABR, F), jnp.float32),
    )(filt_W1, filt_b1, filt_W2, filt_b2)


def _tc_pack(d2, edge_mask, col, row):
    E = d2.shape[0]
    nchunks = E // KM
    CB = 500
    assert nchunks % CB == 0
    invh = np.float32(TAB / CUTOFF)

    def body(d2_ref, em_ref, col_ref, row_ref, out_ref):
        r = jnp.sqrt(d2_ref[:, 0, :])
        b = jnp.minimum(r * invh + 0.5, np.float32(TABR - 1)).astype(jnp.int32)
        b = jnp.where(em_ref[:, 0, :] == 0.0, TABR - 1, b)
        out_ref[:, 0, :] = col_ref[:, 0, :]
        out_ref[:, 1, :] = b
        out_ref[:, 2, :] = row_ref[:, 0, :]

    return pl.pallas_call(
        body,
        grid=(nchunks // CB,),
        in_specs=[
            pl.BlockSpec((CB, 1, KM), lambda e: (e, 0, 0)),
            pl.BlockSpec((CB, 1, KM), lambda e: (e, 0, 0)),
            pl.BlockSpec((CB, 1, KM), lambda e: (e, 0, 0)),
            pl.BlockSpec((CB, 1, KM), lambda e: (e, 0, 0)),
        ],
        out_specs=pl.BlockSpec((CB, 3, KM), lambda e: (e, 0, 0)),
        out_shape=jax.ShapeDtypeStruct((nchunks, 3, KM), jnp.int32),
    )(d2.reshape(nchunks, 1, KM), edge_mask.reshape(nchunks, 1, KM),
      col.reshape(nchunks, 1, KM), row.reshape(nchunks, 1, KM))


def _tc_embed(z2d, emb, in2f_W0):
    N = z2d.shape[0]
    MAXZ, F = emb.shape
    BN = 2000
    assert N % BN == 0

    def body(z_ref, emb_ref, w_ref, h_ref, xf_ref):
        zb = z_ref[:, 0]
        oh = (zb[:, None] == lax.broadcasted_iota(jnp.int32, (1, MAXZ), 1))
        h = jnp.dot(oh.astype(jnp.float32), emb_ref[...],
                    preferred_element_type=jnp.float32)
        h_ref[...] = h
        xf_ref[...] = jnp.dot(h, w_ref[...], preferred_element_type=jnp.float32)

    return pl.pallas_call(
        body,
        grid=(N // BN,),
        in_specs=[
            pl.BlockSpec((BN, 1), lambda n: (n, 0)),
            pl.BlockSpec((MAXZ, F), lambda n: (0, 0)),
            pl.BlockSpec((F, F), lambda n: (0, 0)),
        ],
        out_specs=[
            pl.BlockSpec((BN, F), lambda n: (n, 0)),
            pl.BlockSpec((BN, F), lambda n: (n, 0)),
        ],
        out_shape=[
            jax.ShapeDtypeStruct((N, F), jnp.float32),
            jax.ShapeDtypeStruct((N, F), jnp.float32),
        ],
    )(z2d, emb, in2f_W0)


def _sc_message(xf, tab, pk, zeros_nf):
    N, F = xf.shape
    E = pk.shape[0] * KM
    assert E % KM == 0 and N % 8 == 0
    nchunks = E // KM
    per_w = pl.cdiv(nchunks, NW)
    bases = [-(-(s * N // NSUB) // 8) * 8 for s in range(NSUB)] + [N]
    sizes = [bases[s + 1] - bases[s] for s in range(NSUB)]
    mesh = plsc.VectorSubcoreMesh(core_axis_name="c", subcore_axis_name="s")

    @functools.partial(
        pl.kernel,
        mesh=mesh,
        out_type=jax.ShapeDtypeStruct((NCORE, N, F), jnp.float32),
        scratch_types=[
            pltpu.VMEM((3, 3, KM), jnp.int32),
            pltpu.VMEM((3, KM, F), jnp.float32),
            pltpu.VMEM((3, KM, F), jnp.float32),
            pltpu.VMEM_SHARED((N, F), jnp.float32),
            [pltpu.SemaphoreType.DMA] * 3,
            [pltpu.SemaphoreType.DMA] * 3,
            [pltpu.SemaphoreType.DMA] * 3,
        ],
    )
    def body(xf_hbm, tab_hbm, pk_hbm, zeros_hbm, out_hbm,
             pkv, xjv, wijv, acc, sem_w, sem_g, sem_s):
        c = lax.axis_index("c")
        s = lax.axis_index("s")
        wid = s * NCORE + c
        for ss in range(NSUB):
            @pl.when(s == ss)
            def _():
                pltpu.sync_copy(zeros_hbm.at[pl.ds(bases[ss], sizes[ss])],
                                acc.at[pl.ds(bases[ss], sizes[ss])])
        plsc.subcore_barrier()

        def issue(g, b):
            @pl.when(g >= wid + 3 * NW)
            def _():
                pltpu.make_async_copy(xjv.at[b], acc.at[pkv.at[b, 2]],
                                      sem_s[b]).wait()

            pltpu.sync_copy(pk_hbm.at[g], pkv.at[b])
            pltpu.async_copy(tab_hbm.at[pkv.at[b, 1]], wijv.at[b], sem_w[b])
            pltpu.async_copy(xf_hbm.at[pkv.at[b, 0]], xjv.at[b], sem_g[b])

        issue(wid, 0)
        issue(wid + NW, 1)

        def triple_body(j3, carry):
            for b3 in range(3):
                j = j3 * 3 + b3
                b = b3
                g = j * NW + wid

                @pl.when(g < nchunks)
                def _():
                    pltpu.make_async_copy(
                        tab_hbm.at[pkv.at[b, 1]], wijv.at[b],
                        sem_w[b]).wait()
                    pltpu.make_async_copy(
                        xf_hbm.at[pkv.at[b, 0]], xjv.at[b], sem_g[b]).wait()
                    gn = g + 2 * NW

                    @pl.when(gn < nchunks)
                    def _():
                        issue(gn, (b + 2) % 3)

                    def mul_body(k, mc):
                        for jj in range(F // LANES):
                            sl = pl.ds(jj * LANES, LANES)
                            xjv[b, k, sl] = xjv[b, k, sl] * wijv[b, k, sl]
                        return mc

                    lax.fori_loop(0, KM, mul_body, 0)
                    pltpu.async_copy(xjv.at[b], acc.at[pkv.at[b, 2]],
                                     sem_s[b], add=True)

            return carry

        lax.fori_loop(0, pl.cdiv(per_w, 3), triple_body, 0)
        for b in range(3):
            pltpu.make_async_copy(xjv.at[b], acc.at[pkv.at[b, 2]],
                                  sem_s[b]).wait()
        plsc.subcore_barrier()
        for ss in range(NSUB):
            @pl.when(s == ss)
            def _():
                pltpu.sync_copy(acc.at[pl.ds(bases[ss], sizes[ss])],
                                out_hbm.at[c, pl.ds(bases[ss], sizes[ss])])

    return body(xf, tab, pk, zeros_nf)


def _tc_update(h, mp, W1, b1, W2, b2, in2f_next):
    N, F = h.shape
    BN = 2000
    assert N % BN == 0

    def body(h_ref, mp_ref, w1_ref, b1_ref, w2_ref, b2_ref, wn_ref,
             hn_ref, xf_ref):
        m = mp_ref[0] + mp_ref[1]
        t = _ssp(jnp.dot(m, w1_ref[...], preferred_element_type=jnp.float32)
                 + b1_ref[0][None, :])
        t = jnp.dot(t, w2_ref[...], preferred_element_type=jnp.float32)
        t = t + b2_ref[0][None, :]
        hn = h_ref[...] + t
        hn_ref[...] = hn
        xf_ref[...] = jnp.dot(hn, wn_ref[...], preferred_element_type=jnp.float32)

    return pl.pallas_call(
        body,
        grid=(N // BN,),
        in_specs=[
            pl.BlockSpec((BN, F), lambda n: (n, 0)),
            pl.BlockSpec((NCORE, BN, F), lambda n: (0, n, 0)),
            pl.BlockSpec((F, F), lambda n: (0, 0)),
            pl.BlockSpec((1, F), lambda n: (0, 0)),
            pl.BlockSpec((F, F), lambda n: (0, 0)),
            pl.BlockSpec((1, F), lambda n: (0, 0)),
            pl.BlockSpec((F, F), lambda n: (0, 0)),
        ],
        out_specs=[
            pl.BlockSpec((BN, F), lambda n: (n, 0)),
            pl.BlockSpec((BN, F), lambda n: (n, 0)),
        ],
        out_shape=[
            jax.ShapeDtypeStruct((N, F), jnp.float32),
            jax.ShapeDtypeStruct((N, F), jnp.float32),
        ],
    )(h, mp, W1, b1.reshape(1, -1), W2, b2.reshape(1, -1), in2f_next)


def _tc_update_last(h, mp, W1, b1, W2, b2, atom_mask, node_delta):
    N, F = h.shape
    BN = 2000
    assert N % BN == 0 and BN % MOL_NODES == 0
    mols_per_blk = BN // MOL_NODES

    def body(h_ref, mp_ref, w1_ref, b1_ref, w2_ref, b2_ref, am_ref, nd_ref,
             hm_ref):
        m = mp_ref[0] + mp_ref[1]
        t = _ssp(jnp.dot(m, w1_ref[...], preferred_element_type=jnp.float32)
                 + b1_ref[0][None, :])
        t = jnp.dot(t, w2_ref[...], preferred_element_type=jnp.float32)
        t = t + b2_ref[0][None, :]
        hn = (h_ref[...] + t) * am_ref[...] + nd_ref[0, 0]
        hm_ref[...] = jnp.sum(hn.reshape(mols_per_blk, MOL_NODES, F),
                              axis=1)[None]

    return pl.pallas_call(
        body,
        grid=(N // BN,),
        in_specs=[
            pl.BlockSpec((BN, F), lambda n: (n, 0)),
            pl.BlockSpec((NCORE, BN, F), lambda n: (0, n, 0)),
            pl.BlockSpec((F, F), lambda n: (0, 0)),
            pl.BlockSpec((1, F), lambda n: (0, 0)),
            pl.BlockSpec((F, F), lambda n: (0, 0)),
            pl.BlockSpec((1, F), lambda n: (0, 0)),
            pl.BlockSpec((BN, 1), lambda n: (n, 0)),
            pl.BlockSpec((1, 1), lambda n: (0, 0)),
        ],
        out_specs=pl.BlockSpec((1, mols_per_blk, F), lambda n: (n, 0, 0)),
        out_shape=jax.ShapeDtypeStruct((N // BN, mols_per_blk, F),
                                       jnp.float32),
    )(h, mp, W1, b1.reshape(1, -1), W2, b2.reshape(1, -1), atom_mask,
      node_delta).reshape(N // MOL_NODES, F)


def _tc_readout(hmol, dec_W1, dec_b1, dec_W2, dec_b2, rid2d, sgn2d,
                react_delta):
    B, F = hmol.shape
    H = dec_W1.shape[1]

    def body(hm_ref, w1_ref, b1_ref, w2_ref, b2_ref, rid_ref, sgn_ref,
             rd_ref, out_ref):
        t = _ssp(jnp.dot(hm_ref[...], w1_ref[...],
                         preferred_element_type=jnp.float32)
                 + b1_ref[0][None, :])
        t = jnp.dot(t, w2_ref[...], preferred_element_type=jnp.float32)
        t = t + b2_ref[0][None, :]
        t = t * sgn_ref[...]
        sel = (rid_ref[...] == lax.broadcasted_iota(jnp.int32, (1, N_REACT), 1))
        contrib = sel.astype(jnp.float32) * t
        out_ref[...] = jnp.sum(contrib, axis=0, keepdims=True) + rd_ref[0, 0]

    return pl.pallas_call(
        body,
        grid=(1,),
        in_specs=[
            pl.BlockSpec((B, F), lambda i: (0, 0)),
            pl.BlockSpec((F, H), lambda i: (0, 0)),
            pl.BlockSpec((1, H), lambda i: (0, 0)),
            pl.BlockSpec((H, 1), lambda i: (0, 0)),
            pl.BlockSpec((1, 1), lambda i: (0, 0)),
            pl.BlockSpec((B, 1), lambda i: (0, 0)),
            pl.BlockSpec((B, 1), lambda i: (0, 0)),
            pl.BlockSpec((1, 1), lambda i: (0, 0)),
        ],
        out_specs=pl.BlockSpec((1, N_REACT), lambda i: (0, 0)),
        out_shape=jax.ShapeDtypeStruct((1, N_REACT), jnp.float32),
    )(hmol, dec_W1, dec_b1.reshape(1, -1), dec_W2, dec_b2.reshape(1, -1),
      rid2d, sgn2d, react_delta)


def kernel(z, coord, edge_index, n_nodes, atom_mask, edge_mask, n_reactions,
           reaction_indexes, reaction_indexes_signs, emb, in2f_W, filt_W1,
           filt_b1, filt_W2, filt_b2, f2out_W1, f2out_b1, f2out_W2, f2out_b2,
           dec_W1, dec_b1, dec_W2, dec_b2):
    row = edge_index[0].astype(jnp.int32)
    col = edge_index[1].astype(jnp.int32)
    N, F = coord.shape[0], emb.shape[1]
    NL = in2f_W.shape[0]

    d2 = _sc_edge_dist2(coord, row, col)
    tab = _tc_table(filt_W1, filt_b1, filt_W2, filt_b2)
    pk = _tc_pack(d2, edge_mask, col, row)
    h, xf = _tc_embed(z.astype(jnp.int32).reshape(-1, 1), emb, in2f_W[0])
    zeros_nf = jnp.zeros((N, F), jnp.float32)

    node_delta = (jnp.asarray(n_nodes, jnp.float32)
                  - np.float32(MOL_NODES)).reshape(1, 1)
    react_delta = (jnp.asarray(n_reactions, jnp.float32)
                   - np.float32(N_REACT)).reshape(1, 1)

    hmol = None
    for i in range(NL):
        mp = _sc_message(xf, tab[i], pk, zeros_nf)
        if i + 1 < NL:
            h, xf = _tc_update(h, mp, f2out_W1[i], f2out_b1[i], f2out_W2[i],
                               f2out_b2[i], in2f_W[i + 1])
        else:
            hmol = _tc_update_last(h, mp, f2out_W1[i], f2out_b1[i],
                                   f2out_W2[i], f2out_b2[i], atom_mask,
                                   node_delta)

    rid2d = reaction_indexes.astype(jnp.int32).reshape(-1, 1)
    sgn2d = reaction_indexes_signs.reshape(-1, 1)
    pred = _tc_readout(hmol, dec_W1, dec_b1, dec_W2, dec_b2, rid2d, sgn2d,
                       react_delta)
    return pred[0]
